# baseline (speedup 1.0000x reference)
import functools

import jax
import jax.numpy as jnp
from jax import lax
from jax.experimental import pallas as pl
from jax.experimental.pallas import tpu as pltpu
from jax.experimental.pallas import tpu_sc as plsc

_NC = 2
_NS = 16
_NW = _NC * _NS

_BB = 128
_NBUF = 4
_PADF = 128
_F = 64
_L = 16


def _embed_gather(hist, table_hbm, idx_hbm, out_hbm,
                  idx_v, gbufs, tbufs, gsems, ssems):
    wid = lax.axis_index("s") * _NC + lax.axis_index("c")

    pltpu.sync_copy(idx_hbm.at[:, pl.ds(wid * _BB, _BB)], idx_v)

    vrows = [blk * _L + lax.iota(jnp.int32, _L) for blk in range(_BB // _L)]

    def gather_start(h, slot):
        pltpu.async_copy(table_hbm.at[idx_v.at[h]], gbufs.at[slot],
                         gsems.at[slot])

    def gather_wait(h, slot):
        pltpu.make_async_copy(table_hbm.at[idx_v.at[h]], gbufs.at[slot],
                              gsems.at[slot]).wait()

    def transpose(slot):
        @pl.loop(0, _F)
        def _t(f):
            col = jnp.full((_L,), 0, jnp.int32) + f
            for blk in range(_BB // _L):
                v = plsc.load_gather(gbufs.at[slot], [vrows[blk], col])
                tbufs[slot, f, pl.ds(blk * _L, _L)] = v

    def store_start(h, slot):
        for fi in range(_F // 8):
            pltpu.async_copy(tbufs.at[slot, pl.ds(fi * 8, 8), :],
                             out_hbm.at[h, fi, wid], ssems.at[slot])

    def store_wait(h, slot):
        for fi in range(_F // 8):
            pltpu.make_async_copy(tbufs.at[slot, pl.ds(fi * 8, 8), :],
                                  out_hbm.at[h, fi, wid],
                                  ssems.at[slot]).wait()

    for b in range(_NBUF):
        gather_start(b, b)

    @pl.loop(0, hist - _NBUF, step=_NBUF)
    def _round(j0):
        for b in range(_NBUF):
            gather_wait(j0 + b, b)
            transpose(b)
            store_start(j0 + b, b)
        for b in range(_NBUF):
            store_wait(j0 + b, b)
            gather_start(j0 + b + _NBUF, b)

    last0 = hist - _NBUF
    for b in range(_NBUF):
        gather_wait(last0 + b, b)
        transpose(b)
        store_start(last0 + b, b)
    for b in range(_NBUF):
        store_wait(last0 + b, b)


def kernel(inputs, num_embeddings, features, embedding):
    batch, hist = inputs.shape
    nrows, feat = embedding.shape
    idx_t = inputs.T
    pad_eye = jnp.eye(feat, _PADF, dtype=jnp.float32)
    table_p = lax.dot_general(embedding, pad_eye, (((1,), (0,)), ((), ())),
                              precision=lax.Precision.HIGHEST)

    mesh = plsc.VectorSubcoreMesh(core_axis_name="c", subcore_axis_name="s",
                                  num_cores=_NC, num_subcores=_NS)
    out5 = pl.kernel(
        functools.partial(_embed_gather, hist),
        out_type=jax.ShapeDtypeStruct(
            (hist, feat // 8, batch // _BB, 8, 128), jnp.float32),
        mesh=mesh,
        scratch_types=[
            pltpu.VMEM((hist, _BB), jnp.int32),
            pltpu.VMEM((_NBUF, _BB, _PADF), jnp.float32),
            pltpu.VMEM((_NBUF, feat, _BB), jnp.float32),
            pltpu.SemaphoreType.DMA((_NBUF,)),
            pltpu.SemaphoreType.DMA((_NBUF,)),
        ],
        compiler_params=pltpu.CompilerParams(use_tc_tiling_on_sc=False,
                                             needs_layout_passes=False),
    )(table_p, idx_t)
    return out5.transpose(2, 4, 0, 1, 3).reshape(batch, hist, feat)


# speedup vs baseline: 1.1473x; 1.1473x over previous
import functools

import jax
import jax.numpy as jnp
from jax import lax
from jax.experimental import pallas as pl
from jax.experimental.pallas import tpu as pltpu
from jax.experimental.pallas import tpu_sc as plsc

_NC = 2
_NS = 16
_NW = _NC * _NS

_BB = 128
_NBUF = 4
_PADF = 128
_F = 64
_L = 16


def _embed_gather(hist, table_hbm, idx_hbm, out_hbm,
                  idx_v, gbufs, tbufs, gsems, ssems):
    wid = lax.axis_index("s") * _NC + lax.axis_index("c")

    pltpu.sync_copy(idx_hbm.at[:, pl.ds(wid * _BB, _BB)], idx_v)

    vb = lax.iota(jnp.int32, _L) * _BB

    def gather_start(h, slot):
        pltpu.async_copy(table_hbm.at[idx_v.at[h]], gbufs.at[slot],
                         gsems.at[slot])

    def gather_wait(h, slot):
        pltpu.make_async_copy(table_hbm.at[idx_v.at[h]], gbufs.at[slot],
                              gsems.at[slot]).wait()

    def transpose(slot):
        for f0 in range(_F // _L):
            @pl.loop(0, _BB, unroll=8)
            def _t(bl):
                src = gbufs[slot, bl, pl.ds(f0 * _L, _L)]
                plsc.store_scatter(tbufs.at[slot],
                                   [vb + (bl + f0 * _L * _BB)], src)

    def store_start(h, slot):
        for fi in range(_F // 8):
            pltpu.async_copy(tbufs.at[slot, pl.ds(fi * 8 * _BB, 8 * _BB)],
                             out_hbm.at[h, fi, wid], ssems.at[slot])

    def store_wait(h, slot):
        for fi in range(_F // 8):
            pltpu.make_async_copy(tbufs.at[slot, pl.ds(fi * 8 * _BB, 8 * _BB)],
                                  out_hbm.at[h, fi, wid],
                                  ssems.at[slot]).wait()

    for b in range(_NBUF):
        gather_start(b, b)

    @pl.loop(0, hist - _NBUF, step=_NBUF)
    def _round(j0):
        for b in range(_NBUF):
            gather_wait(j0 + b, b)
            transpose(b)
            store_start(j0 + b, b)
        for b in range(_NBUF):
            store_wait(j0 + b, b)
            gather_start(j0 + b + _NBUF, b)

    last0 = hist - _NBUF
    for b in range(_NBUF):
        gather_wait(last0 + b, b)
        transpose(b)
        store_start(last0 + b, b)
    for b in range(_NBUF):
        store_wait(last0 + b, b)


def kernel(inputs, num_embeddings, features, embedding):
    batch, hist = inputs.shape
    nrows, feat = embedding.shape
    idx_t = inputs.T
    pad_eye = jnp.eye(feat, _PADF, dtype=jnp.float32)
    table_p = lax.dot_general(embedding, pad_eye, (((1,), (0,)), ((), ())),
                              precision=lax.Precision.HIGHEST)

    mesh = plsc.VectorSubcoreMesh(core_axis_name="c", subcore_axis_name="s",
                                  num_cores=_NC, num_subcores=_NS)
    out5 = pl.kernel(
        functools.partial(_embed_gather, hist),
        out_type=jax.ShapeDtypeStruct(
            (hist, feat // 8, batch // _BB, 8 * _BB), jnp.float32),
        mesh=mesh,
        scratch_types=[
            pltpu.VMEM((hist, _BB), jnp.int32),
            pltpu.VMEM((_NBUF, _BB, _PADF), jnp.float32),
            pltpu.VMEM((_NBUF, feat * _BB), jnp.float32),
            pltpu.SemaphoreType.DMA((_NBUF,)),
            pltpu.SemaphoreType.DMA((_NBUF,)),
        ],
        compiler_params=pltpu.CompilerParams(use_tc_tiling_on_sc=False,
                                             needs_layout_passes=False),
    )(table_p, idx_t)
    out5 = out5.reshape(hist, feat // 8, batch // _BB, 8, _BB)
    return out5.transpose(2, 4, 0, 1, 3).reshape(batch, hist, feat)


# speedup vs baseline: 1.8174x; 1.5841x over previous
import functools

import jax
import jax.numpy as jnp
from jax import lax
from jax.experimental import pallas as pl
from jax.experimental.pallas import tpu as pltpu
from jax.experimental.pallas import tpu_sc as plsc

_NC = 2
_NS = 16
_NW = _NC * _NS

_BB = 128
_NBUF = 4
_PADF = 128
_F = 64
_L = 16


def _embed_gather(hist, table_hbm, idx_hbm, out_hbm,
                  idx_v, gbufs, tbufs, gsems, ssems):
    wid = lax.axis_index("s") * _NC + lax.axis_index("c")

    pltpu.sync_copy(idx_hbm.at[:, pl.ds(wid * _BB, _BB)], idx_v)

    vfs = [lax.iota(jnp.int32, _L) + f0 * _L for f0 in range(_F // _L)]
    vzero = jnp.zeros((_L,), jnp.int32)

    def gather_start(h, slot):
        pltpu.async_copy(table_hbm.at[idx_v.at[h]], gbufs.at[slot],
                         gsems.at[slot])

    def gather_wait(h, slot):
        pltpu.make_async_copy(table_hbm.at[idx_v.at[h]], gbufs.at[slot],
                              gsems.at[slot]).wait()

    def transpose(slot):
        for f0 in range(_F // _L):
            vf = vfs[f0]

            @pl.loop(0, _BB, unroll=8)
            def _t(bl):
                src = gbufs[slot, bl, pl.ds(f0 * _L, _L)]
                plsc.store_scatter(tbufs.at[slot], [vf, vzero + bl], src)

    def store_start(h, slot):
        for fi in range(_F // 8):
            pltpu.async_copy(tbufs.at[slot, pl.ds(fi * 8, 8), pl.ds(0, _BB)],
                             out_hbm.at[h, fi, wid], ssems.at[slot])

    def store_wait(h, slot):
        for fi in range(_F // 8):
            pltpu.make_async_copy(
                tbufs.at[slot, pl.ds(fi * 8, 8), pl.ds(0, _BB)],
                out_hbm.at[h, fi, wid], ssems.at[slot]).wait()

    for b in range(_NBUF):
        gather_start(b, b)

    @pl.loop(0, hist - _NBUF, step=_NBUF)
    def _round(j0):
        for b in range(_NBUF):
            gather_wait(j0 + b, b)
            transpose(b)
            store_start(j0 + b, b)
        for b in range(_NBUF):
            store_wait(j0 + b, b)
            gather_start(j0 + b + _NBUF, b)

    last0 = hist - _NBUF
    for b in range(_NBUF):
        gather_wait(last0 + b, b)
        transpose(b)
        store_start(last0 + b, b)
    for b in range(_NBUF):
        store_wait(last0 + b, b)


def kernel(inputs, num_embeddings, features, embedding):
    batch, hist = inputs.shape
    nrows, feat = embedding.shape
    idx_t = inputs.T
    pad_eye = jnp.eye(feat, _PADF, dtype=jnp.float32)
    table_p = lax.dot_general(embedding, pad_eye, (((1,), (0,)), ((), ())),
                              precision=lax.Precision.HIGHEST)

    mesh = plsc.VectorSubcoreMesh(core_axis_name="c", subcore_axis_name="s",
                                  num_cores=_NC, num_subcores=_NS)
    out5 = pl.kernel(
        functools.partial(_embed_gather, hist),
        out_type=jax.ShapeDtypeStruct(
            (hist, feat // 8, batch // _BB, 8, _BB), jnp.float32),
        mesh=mesh,
        scratch_types=[
            pltpu.VMEM((hist, _BB), jnp.int32),
            pltpu.VMEM((_NBUF, _BB, _PADF), jnp.float32),
            pltpu.VMEM((_NBUF, feat, _BB + 1), jnp.float32),
            pltpu.SemaphoreType.DMA((_NBUF,)),
            pltpu.SemaphoreType.DMA((_NBUF,)),
        ],
        compiler_params=pltpu.CompilerParams(use_tc_tiling_on_sc=False,
                                             needs_layout_passes=False),
    )(table_p, idx_t)
    return out5.transpose(2, 4, 0, 1, 3).reshape(batch, hist, feat)


# speedup vs baseline: 1.9441x; 1.0697x over previous
import functools

import jax
import jax.numpy as jnp
from jax import lax
from jax.experimental import pallas as pl
from jax.experimental.pallas import tpu as pltpu
from jax.experimental.pallas import tpu_sc as plsc

_NC = 2
_NS = 16
_NW = _NC * _NS

_CHUNK = 128
_NBUF = 4
_PADF = 128
_VBLK = 8192


def _embed_gather(total_rows, table_hbm, idx_hbm, out_hbm,
                  idx_v, rows_v, gsems, ssems):
    chunks_total = total_rows // _CHUNK
    cpw = chunks_total // _NW
    wid = lax.axis_index("s") * _NC + lax.axis_index("c")
    chunk0 = wid * cpw

    pltpu.sync_copy(idx_hbm.at[pl.ds(chunk0, cpw)], idx_v)

    def gather_start(j, slot):
        pltpu.async_copy(table_hbm.at[idx_v.at[j]], rows_v.at[slot],
                         gsems.at[slot])

    def gather_wait(j, slot):
        pltpu.make_async_copy(table_hbm.at[idx_v.at[j]], rows_v.at[slot],
                              gsems.at[slot]).wait()

    def store_start(j, slot):
        base = (chunk0 + j) * _CHUNK
        pltpu.async_copy(rows_v.at[slot], out_hbm.at[pl.ds(base, _CHUNK)],
                         ssems.at[slot])

    def store_wait(j, slot):
        base = (chunk0 + j) * _CHUNK
        pltpu.make_async_copy(rows_v.at[slot],
                              out_hbm.at[pl.ds(base, _CHUNK)],
                              ssems.at[slot]).wait()

    for b in range(_NBUF):
        gather_start(b, b)

    @pl.loop(0, cpw - _NBUF, step=_NBUF)
    def _round(j0):
        for b in range(_NBUF):
            gather_wait(j0 + b, b)
            store_start(j0 + b, b)
        for b in range(_NBUF):
            store_wait(j0 + b, b)
            gather_start(j0 + b + _NBUF, b)

    last0 = cpw - _NBUF
    for b in range(_NBUF):
        gather_wait(last0 + b, b)
        store_start(last0 + b, b)
    for b in range(_NBUF):
        store_wait(last0 + b, b)


def _pad_rows_body(t_t_ref, eye_ref, out_ref):
    out_ref[...] = lax.dot_general(
        t_t_ref[...], eye_ref[...], (((0,), (0,)), ((), ())),
        precision=lax.Precision.HIGHEST)


def _pad_rows(embedding):
    nrows, feat = embedding.shape
    grid = (nrows + _VBLK - 1) // _VBLK
    eye = jnp.eye(feat, _PADF, dtype=jnp.float32)
    return pl.pallas_call(
        _pad_rows_body,
        grid=(grid,),
        in_specs=[
            pl.BlockSpec((feat, _VBLK), lambda i: (0, i)),
            pl.BlockSpec((feat, _PADF), lambda i: (0, 0)),
        ],
        out_specs=pl.BlockSpec((_VBLK, _PADF), lambda i: (i, 0)),
        out_shape=jax.ShapeDtypeStruct((nrows, _PADF), jnp.float32),
    )(embedding.T, eye)


def kernel(inputs, num_embeddings, features, embedding):
    batch, hist = inputs.shape
    nrows, feat = embedding.shape
    total = batch * hist
    idx2d = inputs.reshape(total // _CHUNK, _CHUNK)
    table_p = _pad_rows(embedding)

    cpw = (total // _CHUNK) // _NW
    mesh = plsc.VectorSubcoreMesh(core_axis_name="c", subcore_axis_name="s",
                                  num_cores=_NC, num_subcores=_NS)
    out = pl.kernel(
        functools.partial(_embed_gather, total),
        out_type=jax.ShapeDtypeStruct((total, _PADF), jnp.float32),
        mesh=mesh,
        scratch_types=[
            pltpu.VMEM((cpw, _CHUNK), jnp.int32),
            pltpu.VMEM((_NBUF, _CHUNK, _PADF), jnp.float32),
            pltpu.SemaphoreType.DMA((_NBUF,)),
            pltpu.SemaphoreType.DMA((_NBUF,)),
        ],
        compiler_params=pltpu.CompilerParams(use_tc_tiling_on_sc=False),
    )(table_p, idx2d)
    return out[:, :feat].reshape(batch, hist, feat)


# speedup vs baseline: 1.9468x; 1.0014x over previous
import functools

import jax
import jax.numpy as jnp
from jax import lax
from jax.experimental import pallas as pl
from jax.experimental.pallas import tpu as pltpu
from jax.experimental.pallas import tpu_sc as plsc

_NC = 2
_NS = 16
_NW = _NC * _NS

_CHUNK = 128
_NBUF = 4
_PADF = 128
_VBLK = 8192


def _embed_gather(total_rows, table_hbm, idx_hbm, out_hbm,
                  idx_v, rows_v, gsems, ssems):
    chunks_total = total_rows // _CHUNK
    cpw = chunks_total // _NW
    wid = lax.axis_index("s") * _NC + lax.axis_index("c")
    chunk0 = wid * cpw

    pltpu.sync_copy(idx_hbm.at[pl.ds(chunk0, cpw)], idx_v)

    def gather_start(j, slot):
        pltpu.async_copy(table_hbm.at[idx_v.at[j]], rows_v.at[slot],
                         gsems.at[slot])

    def gather_wait(j, slot):
        pltpu.make_async_copy(table_hbm.at[idx_v.at[j]], rows_v.at[slot],
                              gsems.at[slot]).wait()

    def store_start(j, slot):
        base = (chunk0 + j) * _CHUNK
        pltpu.async_copy(rows_v.at[slot], out_hbm.at[pl.ds(base, _CHUNK)],
                         ssems.at[slot])

    def store_wait(j, slot):
        base = (chunk0 + j) * _CHUNK
        pltpu.make_async_copy(rows_v.at[slot],
                              out_hbm.at[pl.ds(base, _CHUNK)],
                              ssems.at[slot]).wait()

    for b in range(_NBUF):
        gather_start(b, b)

    @pl.loop(0, cpw - _NBUF, step=_NBUF)
    def _round(j0):
        for b in range(_NBUF):
            gather_wait(j0 + b, b)
            store_start(j0 + b, b)
        for b in range(_NBUF):
            store_wait(j0 + b, b)
            gather_start(j0 + b + _NBUF, b)

    last0 = cpw - _NBUF
    for b in range(_NBUF):
        gather_wait(last0 + b, b)
        store_start(last0 + b, b)
    for b in range(_NBUF):
        store_wait(last0 + b, b)


def _pad_rows_body(t_t_ref, eye_ref, out_ref):
    out_ref[...] = lax.dot_general(
        t_t_ref[...], eye_ref[...], (((0,), (0,)), ((), ())),
        precision=lax.Precision.HIGHEST)


def _pad_rows(embedding):
    nrows, feat = embedding.shape
    grid = (nrows + _VBLK - 1) // _VBLK
    eye = jnp.eye(feat, _PADF, dtype=jnp.float32)
    return pl.pallas_call(
        _pad_rows_body,
        grid=(grid,),
        in_specs=[
            pl.BlockSpec((feat, _VBLK), lambda i: (0, i)),
            pl.BlockSpec((feat, _PADF), lambda i: (0, 0)),
        ],
        out_specs=pl.BlockSpec((_VBLK, _PADF), lambda i: (i, 0)),
        out_shape=jax.ShapeDtypeStruct((nrows, _PADF), jnp.float32),
        compiler_params=pltpu.CompilerParams(
            fuse_transposed_lhs_in_matmul=True),
    )(embedding.T, eye)


def kernel(inputs, num_embeddings, features, embedding):
    batch, hist = inputs.shape
    nrows, feat = embedding.shape
    total = batch * hist
    idx2d = inputs.reshape(total // _CHUNK, _CHUNK)
    table_p = _pad_rows(embedding)

    cpw = (total // _CHUNK) // _NW
    mesh = plsc.VectorSubcoreMesh(core_axis_name="c", subcore_axis_name="s",
                                  num_cores=_NC, num_subcores=_NS)
    out = pl.kernel(
        functools.partial(_embed_gather, total),
        out_type=jax.ShapeDtypeStruct((total, _PADF), jnp.float32),
        mesh=mesh,
        scratch_types=[
            pltpu.VMEM((cpw, _CHUNK), jnp.int32),
            pltpu.VMEM((_NBUF, _CHUNK, _PADF), jnp.float32),
            pltpu.SemaphoreType.DMA((_NBUF,)),
            pltpu.SemaphoreType.DMA((_NBUF,)),
        ],
        compiler_params=pltpu.CompilerParams(use_tc_tiling_on_sc=False),
    )(table_p, idx2d)
    return out[:, :feat].reshape(batch, hist, feat)


# speedup vs baseline: 1.9923x; 1.0234x over previous
import functools

import jax
import jax.numpy as jnp
from jax import lax
from jax.experimental import pallas as pl
from jax.experimental.pallas import tpu as pltpu
from jax.experimental.pallas import tpu_sc as plsc

_NC = 2
_NS = 16
_NW = _NC * _NS

_CHUNK = 128
_NBUF = 4
_PADF = 128


def _embed_gather(total_rows, table_hbm, idx_hbm, out_hbm,
                  idx_v, rows_v, gsems, ssems):
    chunks_total = total_rows // _CHUNK
    cpw = chunks_total // _NW
    wid = lax.axis_index("s") * _NC + lax.axis_index("c")
    chunk0 = wid * cpw

    pltpu.sync_copy(idx_hbm.at[pl.ds(chunk0, cpw)], idx_v)

    def gather_start(j, slot):
        pltpu.async_copy(table_hbm.at[idx_v.at[j]], rows_v.at[slot],
                         gsems.at[slot])

    def gather_wait(j, slot):
        pltpu.make_async_copy(table_hbm.at[idx_v.at[j]], rows_v.at[slot],
                              gsems.at[slot]).wait()

    def store_start(j, slot):
        base = (chunk0 + j) * _CHUNK
        pltpu.async_copy(rows_v.at[slot], out_hbm.at[pl.ds(base, _CHUNK)],
                         ssems.at[slot])

    def store_wait(j, slot):
        base = (chunk0 + j) * _CHUNK
        pltpu.make_async_copy(rows_v.at[slot],
                              out_hbm.at[pl.ds(base, _CHUNK)],
                              ssems.at[slot]).wait()

    for b in range(_NBUF):
        gather_start(b, b)

    @pl.loop(0, cpw - _NBUF, step=_NBUF)
    def _round(j0):
        for b in range(_NBUF):
            gather_wait(j0 + b, b)
            store_start(j0 + b, b)
        for b in range(_NBUF):
            store_wait(j0 + b, b)
            gather_start(j0 + b + _NBUF, b)

    last0 = cpw - _NBUF
    for b in range(_NBUF):
        gather_wait(last0 + b, b)
        store_start(last0 + b, b)
    for b in range(_NBUF):
        store_wait(last0 + b, b)


def kernel(inputs, num_embeddings, features, embedding):
    batch, hist = inputs.shape
    nrows, feat = embedding.shape
    total = batch * hist
    idx2d = inputs.reshape(total // _CHUNK, _CHUNK)
    pad_eye = jnp.eye(feat, _PADF, dtype=jnp.float32)
    table_p = lax.dot_general(embedding, pad_eye, (((1,), (0,)), ((), ())),
                              precision=lax.Precision.HIGHEST)

    cpw = (total // _CHUNK) // _NW
    mesh = plsc.VectorSubcoreMesh(core_axis_name="c", subcore_axis_name="s",
                                  num_cores=_NC, num_subcores=_NS)
    out = pl.kernel(
        functools.partial(_embed_gather, total),
        out_type=jax.ShapeDtypeStruct((total, _PADF), jnp.float32),
        mesh=mesh,
        scratch_types=[
            pltpu.VMEM((cpw, _CHUNK), jnp.int32),
            pltpu.VMEM((_NBUF, _CHUNK, _PADF), jnp.float32),
            pltpu.SemaphoreType.DMA((_NBUF,)),
            pltpu.SemaphoreType.DMA((_NBUF,)),
        ],
        compiler_params=pltpu.CompilerParams(use_tc_tiling_on_sc=False),
    )(table_p, idx2d)
    return out[:, :feat].reshape(batch, hist, feat)
